# baseline (speedup 1.0000x reference)
import functools

import jax
import jax.numpy as jnp
from jax import lax
from jax.experimental import pallas as pl
from jax.experimental.pallas import tpu as pltpu
from jax.experimental.pallas import tpu_sc as plsc

_NUM_CORES = 2
_NUM_SUBCORES = 16
_LANES = 16

_C = 8
_SUB = 3
_NBUF = 2


def _sc_gather_concat_mean(idx_flat, feat_table, bp, kp, d, frac0):
    gran = _NUM_SUBCORES * _C * _NBUF
    n0 = int(round(bp * frac0 / gran)) * gran
    n1 = bp - n0
    npw0 = n0 // _NUM_SUBCORES
    npw1 = n1 // _NUM_SUBCORES
    nchunk0 = npw0 // _C
    nchunk1 = npw1 // _C
    ipc = _C * kp
    subn = ipc // _SUB
    lg = d // _LANES
    inv_k = jnp.float32(1.0 / (kp - 1))

    assert npw0 * _NUM_SUBCORES == n0 and npw1 * _NUM_SUBCORES == n1
    assert nchunk0 * _C == npw0 and nchunk0 % _NBUF == 0
    assert nchunk1 * _C == npw1 and nchunk1 % _NBUF == 0
    assert subn * _SUB == ipc and subn % 8 == 0 and subn <= 128
    assert lg * _LANES == d

    mesh = plsc.VectorSubcoreMesh(
        core_axis_name="c", subcore_axis_name="s",
        num_cores=_NUM_CORES, num_subcores=_NUM_SUBCORES)

    scratch = (
        [pltpu.VMEM((ipc,), jnp.int32) for _ in range(_NBUF)] +
        [pltpu.VMEM((ipc, d), jnp.float32) for _ in range(_NBUF)] +
        [pltpu.VMEM((_C, 2 * d), jnp.float32) for _ in range(_NBUF)] +
        [pltpu.SemaphoreType.DMA for _ in range(3 * _NBUF)]
    )

    @functools.partial(
        pl.kernel,
        mesh=mesh,
        out_type=jax.ShapeDtypeStruct((bp, 2 * d), jnp.float32),
        scratch_types=scratch,
    )
    def k(table_hbm, idx_hbm, out_hbm, *bufs):
        idx_v = bufs[0:_NBUF]
        rows_v = bufs[_NBUF:2 * _NBUF]
        stage_v = bufs[2 * _NBUF:3 * _NBUF]
        si = bufs[3 * _NBUF:4 * _NBUF]
        sr = bufs[4 * _NBUF:5 * _NBUF]
        so = bufs[5 * _NBUF:6 * _NBUF]
        cid = lax.axis_index("c")
        sid = lax.axis_index("s")
        nbase = jnp.where(cid == 0, sid * npw0, n0 + sid * npw1)
        nchunk = jnp.where(cid == 0, nchunk0, nchunk1)
        woff = nbase * kp

        def idx_copy(chunk, buf):
            return pltpu.make_async_copy(
                idx_hbm.at[pl.ds(woff + chunk * ipc, ipc)],
                idx_v[buf], si[buf])

        def gather_copy(buf, s):
            return pltpu.make_async_copy(
                table_hbm.at[idx_v[buf].at[pl.ds(s * subn, subn)]],
                rows_v[buf].at[pl.ds(s * subn, subn)], sr[buf])

        def out_copy(chunk, buf):
            return pltpu.make_async_copy(
                stage_v[buf],
                out_hbm.at[pl.ds(nbase + chunk * _C, _C)], so[buf])

        def prologue():
            pltpu.sync_copy(idx_hbm.at[pl.ds(woff, ipc)], idx_v[0])
            for t in range(1, _NBUF):
                idx_copy(t, t).start()
            for s in range(_SUB):
                gather_copy(0, s).start()
            for t in range(1, _NBUF - 1):
                idx_copy(t, t).wait()
                for s in range(_SUB):
                    gather_copy(t, s).start()

        def reduce_chunk(buf):
            rows = rows_v[buf]
            stage = stage_v[buf]

            def node_body(j, carry):
                rb = j * kp
                for g in range(lg):
                    sl = pl.ds(g * _LANES, _LANES)
                    stage[j, sl] = rows[rb, sl]
                    acc = rows[rb + 1, sl]
                    for kk in range(2, kp):
                        acc = acc + rows[rb + kk, sl]
                    stage[j, pl.ds(d + g * _LANES, _LANES)] = acc * inv_k
                return carry
            lax.fori_loop(0, _C, node_body, 0, unroll=False)

        def loop_body(cg, carry):
            cc0 = cg * _NBUF
            for b in range(_NBUF):
                cc = cc0 + b
                lb = (b + _NBUF - 1) % _NBUF

                @pl.when(cc + _NBUF - 1 < nchunk)
                def _():
                    idx_copy(cc + _NBUF - 1, lb).wait()
                    for s in range(_SUB):
                        gather_copy(lb, s).start()

                for s in range(_SUB):
                    gather_copy(b, s).wait()

                @pl.when(cc + _NBUF < nchunk)
                def _():
                    idx_copy(cc + _NBUF, b).start()

                @pl.when(cc >= _NBUF)
                def _():
                    out_copy(cc - _NBUF, b).wait()

                reduce_chunk(b)
                out_copy(cc, b).start()
            return carry

        @pl.when(nchunk > 0)
        def _():
            prologue()
            lax.fori_loop(0, nchunk // _NBUF, loop_body, 0, unroll=False)
            for t in range(_NBUF):
                out_copy(nchunk - _NBUF + t, t).wait()

    return k(feat_table, idx_flat)


def _linear_body(x_ref, w_ref, b_ref, o_ref):
    acc = jnp.dot(x_ref[...], w_ref[...], preferred_element_type=jnp.float32)
    o_ref[...] = jnp.maximum(acc + b_ref[...], 0.0)


def _tc_linear(x, w1, b1):
    bp, d2 = x.shape
    d = w1.shape[1]
    tb = next(t for t in (1280, 1024, 1152, 640, 512, 128) if bp % t == 0)
    return pl.pallas_call(
        _linear_body,
        grid=(bp // tb,),
        in_specs=[
            pl.BlockSpec((tb, d2), lambda i: (i, 0)),
            pl.BlockSpec((d2, d), lambda i: (0, 0)),
            pl.BlockSpec((1, d), lambda i: (0, 0)),
        ],
        out_specs=pl.BlockSpec((tb, d), lambda i: (i, 0)),
        out_shape=jax.ShapeDtypeStruct((bp, d), jnp.float32),
    )(x, w1, b1.reshape(1, d))


def kernel(nodes, neigh_idx, feat_table, W1, b1):
    b, k = neigh_idx.shape
    d = feat_table.shape[1]
    kp = k + 1

    gran = _NUM_SUBCORES * _C * _NBUF
    bp = -(-b // gran) * gran
    idx = jnp.concatenate(
        [nodes.astype(jnp.int32).reshape(b, 1), neigh_idx.astype(jnp.int32)],
        axis=1)
    idx = jnp.concatenate([idx, jnp.zeros((bp - b, kp), jnp.int32)], axis=0)

    comb = _sc_gather_concat_mean(idx.reshape(-1), feat_table, bp, kp, d,
                                  frac0=0.05)
    out = _tc_linear(comb, W1, b1)
    return out[:b]


# speedup vs baseline: 1.4023x; 1.4023x over previous
import functools

import jax
import jax.numpy as jnp
from jax import lax
from jax.experimental import pallas as pl
from jax.experimental.pallas import tpu as pltpu
from jax.experimental.pallas import tpu_sc as plsc

_NUM_CORES = 2
_NUM_SUBCORES = 16
_LANES = 16

_C = 8
_SUB = 3
_NBUF = 2


def _sc_gather_concat_mean(idx_flat, feat_table, bp, kp, d, frac0):
    gran = _NUM_SUBCORES * _C * _NBUF
    n0 = int(round(bp * frac0 / gran)) * gran
    n1 = bp - n0
    npw0 = n0 // _NUM_SUBCORES
    npw1 = n1 // _NUM_SUBCORES
    nchunk0 = npw0 // _C
    nchunk1 = npw1 // _C
    ipc = _C * kp
    subn = ipc // _SUB
    lg = d // _LANES
    inv_k = jnp.float32(1.0 / (kp - 1))

    assert npw0 * _NUM_SUBCORES == n0 and npw1 * _NUM_SUBCORES == n1
    assert nchunk0 * _C == npw0 and nchunk0 % _NBUF == 0
    assert nchunk1 * _C == npw1 and nchunk1 % _NBUF == 0
    assert subn * _SUB == ipc and subn % 8 == 0 and subn <= 128
    assert lg * _LANES == d

    mesh = plsc.VectorSubcoreMesh(
        core_axis_name="c", subcore_axis_name="s",
        num_cores=_NUM_CORES, num_subcores=_NUM_SUBCORES)

    scratch = (
        [pltpu.VMEM((ipc,), jnp.int32) for _ in range(_NBUF)] +
        [pltpu.VMEM((ipc, d), jnp.float32) for _ in range(_NBUF)] +
        [pltpu.VMEM((_C, 2 * d), jnp.float32) for _ in range(_NBUF)] +
        [pltpu.SemaphoreType.DMA for _ in range(3 * _NBUF)]
    )

    @functools.partial(
        pl.kernel,
        mesh=mesh,
        out_type=jax.ShapeDtypeStruct((bp, 2 * d), jnp.float32),
        scratch_types=scratch,
    )
    def k(table_hbm, idx_hbm, out_hbm, *bufs):
        idx_v = bufs[0:_NBUF]
        rows_v = bufs[_NBUF:2 * _NBUF]
        stage_v = bufs[2 * _NBUF:3 * _NBUF]
        si = bufs[3 * _NBUF:4 * _NBUF]
        sr = bufs[4 * _NBUF:5 * _NBUF]
        so = bufs[5 * _NBUF:6 * _NBUF]
        cid = lax.axis_index("c")
        sid = lax.axis_index("s")
        nbase = jnp.where(cid == 0, sid * npw0, n0 + sid * npw1)
        nchunk = jnp.where(cid == 0, nchunk0, nchunk1)
        woff = nbase * kp

        def idx_copy(chunk, buf):
            return pltpu.make_async_copy(
                idx_hbm.at[pl.ds(woff + chunk * ipc, ipc)],
                idx_v[buf], si[buf])

        def gather_copy(buf, s):
            return pltpu.make_async_copy(
                table_hbm.at[idx_v[buf].at[pl.ds(s * subn, subn)]],
                rows_v[buf].at[pl.ds(s * subn, subn)], sr[buf])

        def out_copy(chunk, buf):
            return pltpu.make_async_copy(
                stage_v[buf],
                out_hbm.at[pl.ds(nbase + chunk * _C, _C)], so[buf])

        def prologue():
            pltpu.sync_copy(idx_hbm.at[pl.ds(woff, ipc)], idx_v[0])
            for t in range(1, _NBUF):
                idx_copy(t, t).start()
            for s in range(_SUB):
                gather_copy(0, s).start()
            for t in range(1, _NBUF - 1):
                idx_copy(t, t).wait()
                for s in range(_SUB):
                    gather_copy(t, s).start()

        def reduce_chunk(buf):
            rows = rows_v[buf]
            stage = stage_v[buf]

            def node_body(j, carry):
                rb = j * kp
                for g in range(lg):
                    sl = pl.ds(g * _LANES, _LANES)
                    stage[j, sl] = rows[rb, sl]
                    acc = rows[rb + 1, sl]
                    for kk in range(2, kp):
                        acc = acc + rows[rb + kk, sl]
                    stage[j, pl.ds(d + g * _LANES, _LANES)] = acc * inv_k
                return carry
            lax.fori_loop(0, _C, node_body, 0, unroll=False)

        def loop_body(cg, carry):
            cc0 = cg * _NBUF
            for b in range(_NBUF):
                cc = cc0 + b
                lb = (b + _NBUF - 1) % _NBUF

                @pl.when(cc + _NBUF - 1 < nchunk)
                def _():
                    idx_copy(cc + _NBUF - 1, lb).wait()
                    for s in range(_SUB):
                        gather_copy(lb, s).start()

                for s in range(_SUB):
                    gather_copy(b, s).wait()

                @pl.when(cc + _NBUF < nchunk)
                def _():
                    idx_copy(cc + _NBUF, b).start()

                @pl.when(cc >= _NBUF)
                def _():
                    out_copy(cc - _NBUF, b).wait()

                reduce_chunk(b)
                out_copy(cc, b).start()
            return carry

        @pl.when(nchunk > 0)
        def _():
            prologue()
            lax.fori_loop(0, nchunk // _NBUF, loop_body, 0, unroll=False)
            for t in range(_NBUF):
                out_copy(nchunk - _NBUF + t, t).wait()

    return k(feat_table, idx_flat)


def _linear_body(x_ref, w_ref, b_ref, o_ref):
    acc = jnp.dot(x_ref[...], w_ref[...], preferred_element_type=jnp.float32)
    o_ref[...] = jnp.maximum(acc + b_ref[...], 0.0)


def _tc_linear(x, w1, b1, nrows):
    bp, d2 = x.shape
    d = w1.shape[1]
    tb = next(t for t in (1280, 1024, 1152, 640, 512, 128) if bp % t == 0)
    return pl.pallas_call(
        _linear_body,
        grid=(bp // tb,),
        in_specs=[
            pl.BlockSpec((tb, d2), lambda i: (i, 0)),
            pl.BlockSpec((d2, d), lambda i: (0, 0)),
            pl.BlockSpec((1, d), lambda i: (0, 0)),
        ],
        out_specs=pl.BlockSpec((tb, d), lambda i: (i, 0)),
        out_shape=jax.ShapeDtypeStruct((nrows, d), jnp.float32),
    )(x, w1, b1.reshape(1, d))


def kernel(nodes, neigh_idx, feat_table, W1, b1):
    b, k = neigh_idx.shape
    d = feat_table.shape[1]
    kp = k + 1

    gran = _NUM_SUBCORES * _C * _NBUF
    bp = -(-b // gran) * gran
    idx = jnp.concatenate(
        [nodes.astype(jnp.int32).reshape(b, 1), neigh_idx.astype(jnp.int32)],
        axis=1)
    idx = jnp.concatenate([idx, jnp.zeros((bp - b, kp), jnp.int32)], axis=0)

    comb = _sc_gather_concat_mean(idx.reshape(-1), feat_table, bp, kp, d,
                                  frac0=0.95)
    return _tc_linear(comb, W1, b1, b)
